# baseline (speedup 1.0000x reference)
import jax
import jax.numpy as jnp
from jax import lax
from jax.experimental import pallas as pl
from jax.experimental.pallas import tpu as pltpu
from jax.experimental.pallas import tpu_sc as plsc

N_TOK = 64
D = 768
E = 64
H = 768

_F32 = jnp.float32
_HI = lax.Precision.HIGHEST


def _router_body(x_ref, rw_ref, gi0_ref, gi1_ref, gw0_ref, gw1_ref, aux_ref):
    x = x_ref[...]
    rw = rw_ref[...]
    logits = lax.dot_general(x, rw, (((1,), (1,)), ((), ())),
                             preferred_element_type=_F32, precision=_HI)
    m = jnp.max(logits, axis=1, keepdims=True)
    el = jnp.exp(logits - m)
    probs = el / jnp.sum(el, axis=1, keepdims=True)

    lane = lax.broadcasted_iota(jnp.int32, (N_TOK, E), 1)
    v0 = jnp.max(probs, axis=1, keepdims=True)
    i0 = jnp.min(jnp.where(probs == v0, lane, E), axis=1, keepdims=True)
    probs2 = jnp.where(lane == i0, -1.0, probs)
    v1 = jnp.max(probs2, axis=1, keepdims=True)
    i1 = jnp.min(jnp.where(probs2 == v1, lane, E), axis=1, keepdims=True)

    s = v0 + v1
    gi0_ref[...] = i0
    gi1_ref[...] = i1
    gw0_ref[...] = v0 / s
    gw1_ref[...] = v1 / s

    one0 = (lane == i0).astype(_F32)
    f = jnp.sum(one0, axis=0, keepdims=True) / N_TOK
    P = jnp.sum(probs, axis=0, keepdims=True) / N_TOK
    aux_ref[...] = jnp.sum(f * P, keepdims=True) * E


def _router(x2d, router_w):
    return pl.pallas_call(
        _router_body,
        out_shape=(
            jax.ShapeDtypeStruct((N_TOK, 1), jnp.int32),
            jax.ShapeDtypeStruct((N_TOK, 1), jnp.int32),
            jax.ShapeDtypeStruct((N_TOK, 1), _F32),
            jax.ShapeDtypeStruct((N_TOK, 1), _F32),
            jax.ShapeDtypeStruct((1, 1), _F32),
        ),
    )(x2d, router_w)


_L = 16
_NV = E // _L


def _dispatch_body(gi0_hbm, gi1_hbm, ids_hbm, idx_v, flag_v, ids_v):
    cid = lax.axis_index("c")
    sid = lax.axis_index("s")

    @pl.when((cid == 0) & (sid == 0))
    def _():
        pltpu.sync_copy(gi0_hbm, idx_v.at[pl.ds(0, N_TOK)])
        pltpu.sync_copy(gi1_hbm, idx_v.at[pl.ds(N_TOK, N_TOK)])
        zeros = jnp.zeros((_L,), jnp.int32)
        ones = jnp.ones((_L,), jnp.int32)
        for j in range(_NV):
            flag_v[pl.ds(_L * j, _L)] = zeros
        for j in range(2 * N_TOK // _L):
            idx = idx_v[pl.ds(_L * j, _L)]
            idx = jnp.minimum(jnp.maximum(idx, 0), E - 1)
            plsc.store_scatter(flag_v, [idx], ones, mask=ones > 0)
        iota = lax.iota(jnp.int32, _L)
        carry = jnp.int32(0)
        last = jnp.int32(0)
        pos = []
        flags = []
        for j in range(_NV):
            t = flag_v[pl.ds(_L * j, _L)]
            incl = plsc.cumsum(t)
            pos.append(incl - t + carry)
            flags.append(t)
            carry = carry + jnp.sum(t)
            cand = jnp.where(t > 0, iota + _L * j, -1)
            last = jnp.maximum(last, jnp.max(cand))
        for j in range(_NV):
            ids_v[pl.ds(_L * j, _L)] = jnp.full((_L,), last, jnp.int32)
        for j in range(_NV):
            plsc.store_scatter(ids_v, [pos[j]], iota + _L * j,
                               mask=flags[j] > 0)
        pltpu.sync_copy(ids_v, ids_hbm)


def _dispatch(gi0, gi1):
    mesh = plsc.VectorSubcoreMesh(core_axis_name="c", subcore_axis_name="s")
    f = pl.kernel(
        _dispatch_body,
        out_type=jax.ShapeDtypeStruct((E,), jnp.int32),
        mesh=mesh,
        scratch_types=[
            pltpu.VMEM((2 * N_TOK,), jnp.int32),
            pltpu.VMEM((E,), jnp.int32),
            pltpu.VMEM((E,), jnp.int32),
        ],
        compiler_params=pltpu.CompilerParams(needs_layout_passes=False),
    )
    return f(gi0, gi1)


def _expert_body(ids_ref, x_ref, gi0_ref, gi1_ref, gw0_ref, gw1_ref,
                 w1_ref, w2_ref, w3_ref, out_ref):
    i = pl.program_id(0)
    e = ids_ref[i]
    prev = ids_ref[jnp.maximum(i - 1, 0)]
    active = (i == 0) | (e > prev)

    @pl.when(i == 0)
    def _():
        out_ref[...] = jnp.zeros_like(out_ref)

    @pl.when(active)
    def _():
        x = x_ref[...]
        w1 = w1_ref[0]
        w2 = w2_ref[0]
        w3 = w3_ref[0]
        h1 = lax.dot_general(x, w1, (((1,), (1,)), ((), ())),
                             preferred_element_type=_F32)
        h2 = lax.dot_general(x, w2, (((1,), (1,)), ((), ())),
                             preferred_element_type=_F32)
        h = h1 * jax.nn.sigmoid(h1) * h2
        y = lax.dot_general(h, w3, (((1,), (1,)), ((), ())),
                            preferred_element_type=_F32)
        g = (jnp.where(gi0_ref[...] == e, gw0_ref[...], 0.0)
             + jnp.where(gi1_ref[...] == e, gw1_ref[...], 0.0))
        out_ref[...] += g * y


def _experts(ids, x2d, gi0, gi1, gw0, gw1, w1, w2, w3):
    full = lambda i, ids_ref: (0, 0)
    wmap = lambda i, ids_ref: (ids_ref[i], 0, 0)
    grid_spec = pltpu.PrefetchScalarGridSpec(
        num_scalar_prefetch=1,
        grid=(E,),
        in_specs=[
            pl.BlockSpec((N_TOK, D), full),
            pl.BlockSpec((N_TOK, 1), full),
            pl.BlockSpec((N_TOK, 1), full),
            pl.BlockSpec((N_TOK, 1), full),
            pl.BlockSpec((N_TOK, 1), full),
            pl.BlockSpec((1, H, D), wmap),
            pl.BlockSpec((1, H, D), wmap),
            pl.BlockSpec((1, D, H), wmap),
        ],
        out_specs=pl.BlockSpec((N_TOK, D), full),
    )
    return pl.pallas_call(
        _expert_body,
        grid_spec=grid_spec,
        out_shape=jax.ShapeDtypeStruct((N_TOK, D), _F32),
    )(ids, x2d, gi0, gi1, gw0, gw1, w1, w2, w3)


def kernel(x, router_w, w1, w2, w3):
    B, T, Dm = x.shape
    x2d = x.reshape(B * T, Dm)
    gi0, gi1, gw0, gw1, aux = _router(x2d, router_w)
    ids = jnp.arange(E, dtype=jnp.int32)
    out = _experts(ids, x2d, gi0, gi1, gw0, gw1, w1, w2, w3)
    return out.reshape(B, T, Dm), aux[0, 0]


# speedup vs baseline: 1.0485x; 1.0485x over previous
import jax
import jax.numpy as jnp
from jax import lax
from jax.experimental import pallas as pl
from jax.experimental.pallas import tpu as pltpu
from jax.experimental.pallas import tpu_sc as plsc

N_TOK = 64
D = 768
E = 64
H = 768

_F32 = jnp.float32
_HI = lax.Precision.HIGHEST


def _router_body(x_ref, rw_ref, gi0_ref, gi1_ref, gw0_ref, gw1_ref, aux_ref):
    x = x_ref[...]
    rw = rw_ref[...]
    logits = lax.dot_general(x, rw, (((1,), (1,)), ((), ())),
                             preferred_element_type=_F32, precision=_HI)
    m = jnp.max(logits, axis=1, keepdims=True)
    el = jnp.exp(logits - m)
    probs = el / jnp.sum(el, axis=1, keepdims=True)

    lane = lax.broadcasted_iota(jnp.int32, (N_TOK, E), 1)
    v0 = jnp.max(probs, axis=1, keepdims=True)
    i0 = jnp.min(jnp.where(probs == v0, lane, E), axis=1, keepdims=True)
    probs2 = jnp.where(lane == i0, -1.0, probs)
    v1 = jnp.max(probs2, axis=1, keepdims=True)
    i1 = jnp.min(jnp.where(probs2 == v1, lane, E), axis=1, keepdims=True)

    s = v0 + v1
    gi0_ref[...] = i0
    gi1_ref[...] = i1
    gw0_ref[...] = v0 / s
    gw1_ref[...] = v1 / s

    one0 = (lane == i0).astype(_F32)
    f = jnp.sum(one0, axis=0, keepdims=True) / N_TOK
    P = jnp.sum(probs, axis=0, keepdims=True) / N_TOK
    aux_ref[...] = jnp.sum(f * P, keepdims=True) * E


def _router(x2d, router_w):
    return pl.pallas_call(
        _router_body,
        out_shape=(
            jax.ShapeDtypeStruct((N_TOK, 1), jnp.int32),
            jax.ShapeDtypeStruct((N_TOK, 1), jnp.int32),
            jax.ShapeDtypeStruct((N_TOK, 1), _F32),
            jax.ShapeDtypeStruct((N_TOK, 1), _F32),
            jax.ShapeDtypeStruct((1, 1), _F32),
        ),
    )(x2d, router_w)


_L = 16
_NV = E // _L


def _dispatch_body(gi0_hbm, gi1_hbm, ids_hbm, cnt_hbm, idx_v, flag_v, ids_v,
                   cnt_v):
    cid = lax.axis_index("c")
    sid = lax.axis_index("s")

    @pl.when((cid == 0) & (sid == 0))
    def _():
        pltpu.sync_copy(gi0_hbm, idx_v.at[pl.ds(0, N_TOK)])
        pltpu.sync_copy(gi1_hbm, idx_v.at[pl.ds(N_TOK, N_TOK)])
        zeros = jnp.zeros((_L,), jnp.int32)
        ones = jnp.ones((_L,), jnp.int32)
        for j in range(_NV):
            flag_v[pl.ds(_L * j, _L)] = zeros
        for j in range(2 * N_TOK // _L):
            idx = idx_v[pl.ds(_L * j, _L)]
            idx = jnp.minimum(jnp.maximum(idx, 0), E - 1)
            plsc.store_scatter(flag_v, [idx], ones, mask=ones > 0)
        iota = lax.iota(jnp.int32, _L)
        carry = jnp.int32(0)
        last = jnp.int32(0)
        pos = []
        flags = []
        for j in range(_NV):
            t = flag_v[pl.ds(_L * j, _L)]
            incl = plsc.cumsum(t)
            pos.append(incl - t + carry)
            flags.append(t)
            carry = carry + jnp.sum(t)
            cand = jnp.where(t > 0, iota + _L * j, -1)
            last = jnp.maximum(last, jnp.max(cand))
        for j in range(_NV):
            ids_v[pl.ds(_L * j, _L)] = jnp.full((_L,), last, jnp.int32)
        for j in range(_NV):
            plsc.store_scatter(ids_v, [pos[j]], iota + _L * j,
                               mask=flags[j] > 0)
        cnt_v[...] = jnp.full((_L,), carry, jnp.int32)
        pltpu.sync_copy(ids_v, ids_hbm)
        pltpu.sync_copy(cnt_v, cnt_hbm)


def _dispatch(gi0, gi1):
    mesh = plsc.VectorSubcoreMesh(core_axis_name="c", subcore_axis_name="s")
    f = pl.kernel(
        _dispatch_body,
        out_type=(
            jax.ShapeDtypeStruct((E,), jnp.int32),
            jax.ShapeDtypeStruct((_L,), jnp.int32),
        ),
        mesh=mesh,
        scratch_types=[
            pltpu.VMEM((2 * N_TOK,), jnp.int32),
            pltpu.VMEM((E,), jnp.int32),
            pltpu.VMEM((E,), jnp.int32),
            pltpu.VMEM((_L,), jnp.int32),
        ],
        compiler_params=pltpu.CompilerParams(needs_layout_passes=False),
    )
    return f(gi0, gi1)


def _expert_body(ids_ref, x_ref, gi0_ref, gi1_ref, gw0_ref, gw1_ref,
                 w1_ref, w2_ref, w3_ref, out_ref):
    i = pl.program_id(0)
    e = ids_ref[i]

    @pl.when(i == 0)
    def _():
        out_ref[...] = jnp.zeros_like(out_ref)

    x = x_ref[...]
    w1 = w1_ref[0]
    w2 = w2_ref[0]
    w3 = w3_ref[0]
    h1 = lax.dot_general(x, w1, (((1,), (1,)), ((), ())),
                         preferred_element_type=_F32)
    h2 = lax.dot_general(x, w2, (((1,), (1,)), ((), ())),
                         preferred_element_type=_F32)
    h = h1 * jax.nn.sigmoid(h1) * h2
    y = lax.dot_general(h, w3, (((1,), (1,)), ((), ())),
                        preferred_element_type=_F32)
    g = (jnp.where(gi0_ref[...] == e, gw0_ref[...], 0.0)
         + jnp.where(gi1_ref[...] == e, gw1_ref[...], 0.0))
    out_ref[...] += g * y


def _experts(cnt, ids, x2d, gi0, gi1, gw0, gw1, w1, w2, w3):
    full = lambda i, ids_ref: (0, 0)
    wmap = lambda i, ids_ref: (ids_ref[i], 0, 0)
    grid_spec = pltpu.PrefetchScalarGridSpec(
        num_scalar_prefetch=1,
        grid=(cnt,),
        in_specs=[
            pl.BlockSpec((N_TOK, D), full),
            pl.BlockSpec((N_TOK, 1), full),
            pl.BlockSpec((N_TOK, 1), full),
            pl.BlockSpec((N_TOK, 1), full),
            pl.BlockSpec((N_TOK, 1), full),
            pl.BlockSpec((1, H, D), wmap),
            pl.BlockSpec((1, H, D), wmap),
            pl.BlockSpec((1, D, H), wmap),
        ],
        out_specs=pl.BlockSpec((N_TOK, D), full),
    )
    return pl.pallas_call(
        _expert_body,
        grid_spec=grid_spec,
        out_shape=jax.ShapeDtypeStruct((N_TOK, D), _F32),
    )(ids, x2d, gi0, gi1, gw0, gw1, w1, w2, w3)


def kernel(x, router_w, w1, w2, w3):
    B, T, Dm = x.shape
    x2d = x.reshape(B * T, Dm)
    gi0, gi1, gw0, gw1, aux = _router(x2d, router_w)
    ids, cnt_arr = _dispatch(gi0.reshape(N_TOK), gi1.reshape(N_TOK))
    out = _experts(cnt_arr[0], ids, x2d, gi0, gi1, gw0, gw1, w1, w2, w3)
    return out.reshape(B, T, Dm), aux[0, 0]


# speedup vs baseline: 1.1728x; 1.1186x over previous
import jax
import jax.numpy as jnp
from jax import lax
from jax.experimental import pallas as pl
from jax.experimental.pallas import tpu as pltpu
from jax.experimental.pallas import tpu_sc as plsc

N_TOK = 64
D = 768
E = 64
H = 768

_F32 = jnp.float32
_HI = lax.Precision.HIGHEST


def _router_body(x_ref, rw_ref, gi0_ref, gi1_ref, gw0_ref, gw1_ref, aux_ref,
                 ids_ref, cnt_ref):
    x = x_ref[...]
    rw = rw_ref[...]
    logits = lax.dot_general(x, rw, (((1,), (1,)), ((), ())),
                             preferred_element_type=_F32, precision=_HI)
    m = jnp.max(logits, axis=1, keepdims=True)
    el = jnp.exp(logits - m)
    probs = el / jnp.sum(el, axis=1, keepdims=True)

    lane = lax.broadcasted_iota(jnp.int32, (N_TOK, E), 1)
    v0 = jnp.max(probs, axis=1, keepdims=True)
    i0 = jnp.min(jnp.where(probs == v0, lane, E), axis=1, keepdims=True)
    probs2 = jnp.where(lane == i0, -1.0, probs)
    v1 = jnp.max(probs2, axis=1, keepdims=True)
    i1 = jnp.min(jnp.where(probs2 == v1, lane, E), axis=1, keepdims=True)

    s = v0 + v1
    gi0_ref[...] = i0
    gi1_ref[...] = i1
    gw0_ref[...] = v0 / s
    gw1_ref[...] = v1 / s

    one0 = (lane == i0).astype(_F32)
    f = jnp.sum(one0, axis=0, keepdims=True) / N_TOK
    P = jnp.sum(probs, axis=0, keepdims=True) / N_TOK
    aux_ref[...] = jnp.sum(f * P, keepdims=True) * E

    one1 = (lane == i1).astype(_F32)
    t = ((jnp.sum(one0 + one1, axis=0, keepdims=True)) > 0).astype(_F32)
    a = lax.broadcasted_iota(jnp.int32, (E, E), 0)
    b = lax.broadcasted_iota(jnp.int32, (E, E), 1)
    tri = (a < b).astype(_F32)
    pos = lax.dot_general(t, tri, (((1,), (0,)), ((), ())),
                          preferred_element_type=_F32)
    cnt = jnp.sum(t, axis=1, keepdims=True)
    lanef = lane[:1].astype(_F32)
    last = jnp.max(t * lanef, axis=1, keepdims=True)
    rowi = lax.broadcasted_iota(jnp.int32, (E, E), 0).astype(_F32)
    hit = (pos == rowi).astype(_F32) * t
    idsf = jnp.sum(hit * lanef, axis=1, keepdims=True)
    slot = lax.broadcasted_iota(jnp.int32, (E, 1), 0).astype(_F32)
    idsf = jnp.where(slot < cnt, idsf, last)
    ids_ref[...] = idsf.astype(jnp.int32)
    cnt_ref[...] = cnt.astype(jnp.int32)


def _router(x2d, router_w):
    return pl.pallas_call(
        _router_body,
        out_shape=(
            jax.ShapeDtypeStruct((N_TOK, 1), jnp.int32),
            jax.ShapeDtypeStruct((N_TOK, 1), jnp.int32),
            jax.ShapeDtypeStruct((N_TOK, 1), _F32),
            jax.ShapeDtypeStruct((N_TOK, 1), _F32),
            jax.ShapeDtypeStruct((1, 1), _F32),
            jax.ShapeDtypeStruct((E, 1), jnp.int32),
            jax.ShapeDtypeStruct((1, 1), jnp.int32),
        ),
    )(x2d, router_w)


_L = 16
_NV = E // _L


def _dispatch_body(gi0_hbm, gi1_hbm, ids_hbm, cnt_hbm, idx_v, flag_v, ids_v,
                   cnt_v):
    cid = lax.axis_index("c")
    sid = lax.axis_index("s")

    @pl.when((cid == 0) & (sid == 0))
    def _():
        pltpu.sync_copy(gi0_hbm, idx_v.at[pl.ds(0, N_TOK)])
        pltpu.sync_copy(gi1_hbm, idx_v.at[pl.ds(N_TOK, N_TOK)])
        zeros = jnp.zeros((_L,), jnp.int32)
        ones = jnp.ones((_L,), jnp.int32)
        for j in range(_NV):
            flag_v[pl.ds(_L * j, _L)] = zeros
        for j in range(2 * N_TOK // _L):
            idx = idx_v[pl.ds(_L * j, _L)]
            idx = jnp.minimum(jnp.maximum(idx, 0), E - 1)
            plsc.store_scatter(flag_v, [idx], ones, mask=ones > 0)
        iota = lax.iota(jnp.int32, _L)
        carry = jnp.int32(0)
        last = jnp.int32(0)
        pos = []
        flags = []
        for j in range(_NV):
            t = flag_v[pl.ds(_L * j, _L)]
            incl = plsc.cumsum(t)
            pos.append(incl - t + carry)
            flags.append(t)
            carry = carry + jnp.sum(t)
            cand = jnp.where(t > 0, iota + _L * j, -1)
            last = jnp.maximum(last, jnp.max(cand))
        for j in range(_NV):
            ids_v[pl.ds(_L * j, _L)] = jnp.full((_L,), last, jnp.int32)
        for j in range(_NV):
            plsc.store_scatter(ids_v, [pos[j]], iota + _L * j,
                               mask=flags[j] > 0)
        cnt_v[...] = jnp.full((_L,), carry, jnp.int32)
        pltpu.sync_copy(ids_v, ids_hbm)
        pltpu.sync_copy(cnt_v, cnt_hbm)


def _dispatch(gi0, gi1):
    mesh = plsc.VectorSubcoreMesh(core_axis_name="c", subcore_axis_name="s")
    f = pl.kernel(
        _dispatch_body,
        out_type=(
            jax.ShapeDtypeStruct((E,), jnp.int32),
            jax.ShapeDtypeStruct((_L,), jnp.int32),
        ),
        mesh=mesh,
        scratch_types=[
            pltpu.VMEM((2 * N_TOK,), jnp.int32),
            pltpu.VMEM((E,), jnp.int32),
            pltpu.VMEM((E,), jnp.int32),
            pltpu.VMEM((_L,), jnp.int32),
        ],
        compiler_params=pltpu.CompilerParams(needs_layout_passes=False),
    )
    return f(gi0, gi1)


def _expert_body(ids_ref, x_ref, gi0_ref, gi1_ref, gw0_ref, gw1_ref,
                 w1_ref, w2_ref, w3_ref, out_ref):
    i = pl.program_id(0)
    e = ids_ref[i]

    @pl.when(i == 0)
    def _():
        out_ref[...] = jnp.zeros_like(out_ref)

    x = x_ref[...]
    w1 = w1_ref[0]
    w2 = w2_ref[0]
    w3 = w3_ref[0]
    h1 = lax.dot_general(x, w1, (((1,), (1,)), ((), ())),
                         preferred_element_type=_F32)
    h2 = lax.dot_general(x, w2, (((1,), (1,)), ((), ())),
                         preferred_element_type=_F32)
    h = h1 * jax.nn.sigmoid(h1) * h2
    y = lax.dot_general(h, w3, (((1,), (1,)), ((), ())),
                        preferred_element_type=_F32)
    g = (jnp.where(gi0_ref[...] == e, gw0_ref[...], 0.0)
         + jnp.where(gi1_ref[...] == e, gw1_ref[...], 0.0))
    out_ref[...] += g * y


def _experts(cnt, ids, x2d, gi0, gi1, gw0, gw1, w1, w2, w3):
    full = lambda i, ids_ref: (0, 0)
    wmap = lambda i, ids_ref: (ids_ref[i], 0, 0)
    grid_spec = pltpu.PrefetchScalarGridSpec(
        num_scalar_prefetch=1,
        grid=(cnt,),
        in_specs=[
            pl.BlockSpec((N_TOK, D), full),
            pl.BlockSpec((N_TOK, 1), full),
            pl.BlockSpec((N_TOK, 1), full),
            pl.BlockSpec((N_TOK, 1), full),
            pl.BlockSpec((N_TOK, 1), full),
            pl.BlockSpec((1, H, D), wmap),
            pl.BlockSpec((1, H, D), wmap),
            pl.BlockSpec((1, D, H), wmap),
        ],
        out_specs=pl.BlockSpec((N_TOK, D), full),
    )
    return pl.pallas_call(
        _expert_body,
        grid_spec=grid_spec,
        out_shape=jax.ShapeDtypeStruct((N_TOK, D), _F32),
    )(ids, x2d, gi0, gi1, gw0, gw1, w1, w2, w3)


def kernel(x, router_w, w1, w2, w3):
    B, T, Dm = x.shape
    x2d = x.reshape(B * T, Dm)
    gi0, gi1, gw0, gw1, aux, ids2d, cnt2d = _router(x2d, router_w)
    out = _experts(cnt2d[0, 0], ids2d.reshape(E), x2d,
                   gi0, gi1, gw0, gw1, w1, w2, w3)
    return out.reshape(B, T, Dm), aux[0, 0]
